# Initial kernel scaffold; baseline (speedup 1.0000x reference)
import jax
import jax.numpy as jnp
from jax.experimental import pallas as pl


def kernel(x, edge_index, W1, b1, W2, b2):
    raise NotImplementedError("write your pallas kernel here")


# speedup vs baseline: 11.1652x; 11.1652x over previous
import functools

import jax
import jax.numpy as jnp
from jax import lax
from jax.experimental import pallas as pl
from jax.experimental.pallas import tpu as pltpu
from jax.experimental.pallas import tpu_sc as plsc

N = 10000
D = 128
E = 320000
NC = 2
NS = 16
NW = NC * NS
B = 128
NBATCH = 79
EPAD = NW * NBATCH * B
NACC = 10240
ZROWS = NACC // NS
DEGW = 16

_mesh = plsc.VectorSubcoreMesh(
    core_axis_name="c", subcore_axis_name="s", num_cores=NC, num_subcores=NS
)


@functools.partial(
    pl.kernel,
    out_type=jax.ShapeDtypeStruct((NC, NACC, DEGW), jnp.float32),
    mesh=_mesh,
    scratch_types=[
        pltpu.VMEM((B,), jnp.int32),
        pltpu.VMEM((B, DEGW), jnp.float32),
        pltpu.VMEM_SHARED((NACC, DEGW), jnp.float32),
    ],
)
def _deg_kernel(dst_hbm, out_hbm, didx, ones, acc):
    c = lax.axis_index("c")
    s = lax.axis_index("s")
    w = s * NC + c

    def fill(val):
        def body(j, _):
            ones[j, :] = jnp.full((DEGW,), val, jnp.float32)
            return 0
        lax.fori_loop(0, B, body, 0)

    fill(0.0)
    for k in range(ZROWS // B):
        base = pl.multiple_of(s * ZROWS + k * B, B)
        pltpu.sync_copy(ones, acc.at[pl.ds(base, B)])
    fill(1.0)
    plsc.subcore_barrier()

    def body(b, _):
        off = pl.multiple_of((w * NBATCH + b) * B, B)
        pltpu.sync_copy(dst_hbm.at[pl.ds(off, B)], didx)
        pltpu.sync_copy(ones, acc.at[didx], add=True)
        return 0

    lax.fori_loop(0, NBATCH, body, 0)
    plsc.subcore_barrier()
    for k in range(ZROWS // B):
        base = pl.multiple_of(s * ZROWS + k * B, B)
        pltpu.sync_copy(acc.at[pl.ds(base, B)], ones)
        pltpu.sync_copy(ones, out_hbm.at[c, pl.ds(base, B)])


@functools.partial(
    pl.kernel,
    out_type=jax.ShapeDtypeStruct((NC, NACC, D), jnp.float32),
    mesh=_mesh,
    scratch_types=[
        pltpu.VMEM((B,), jnp.int32),
        pltpu.VMEM((B,), jnp.int32),
        pltpu.VMEM((B, D), jnp.float32),
        pltpu.VMEM_SHARED((NACC, D), jnp.float32),
        pltpu.SemaphoreType.DMA,
    ],
)
def _agg_kernel(hp_hbm, src_hbm, dst_hbm, out_hbm, sidx, didx, rows, acc, sem):
    c = lax.axis_index("c")
    s = lax.axis_index("s")
    w = s * NC + c

    def zero(i, _):
        for j in range(D // 16):
            rows[i, pl.ds(j * 16, 16)] = jnp.zeros((16,), jnp.float32)
        return 0

    lax.fori_loop(0, B, zero, 0)
    for k in range(ZROWS // B):
        base = pl.multiple_of(s * ZROWS + k * B, B)
        pltpu.sync_copy(rows, acc.at[pl.ds(base, B)])
    plsc.subcore_barrier()

    def body(b, _):
        off = pl.multiple_of((w * NBATCH + b) * B, B)
        pltpu.sync_copy(src_hbm.at[pl.ds(off, B)], sidx)
        pltpu.sync_copy(dst_hbm.at[pl.ds(off, B)], didx)
        pltpu.async_copy(hp_hbm.at[sidx], rows, sem).wait()
        pltpu.sync_copy(rows, acc.at[didx], add=True)
        return 0

    lax.fori_loop(0, NBATCH, body, 0)
    plsc.subcore_barrier()
    for k in range(ZROWS // B):
        base = pl.multiple_of(s * ZROWS + k * B, B)
        pltpu.sync_copy(acc.at[pl.ds(base, B)], rows)
        pltpu.sync_copy(rows, out_hbm.at[c, pl.ds(base, B)])


_RB = 1000


def _tc1_body(x_ref, w_ref, deg_ref, hp_ref, dinv_ref):
    deg = deg_ref[0] + deg_ref[1] + 1.0
    dinv = lax.rsqrt(deg)
    dinv_ref[...] = dinv
    scale = dinv[:, 0:1]
    hp_ref[...] = (
        jnp.dot(x_ref[...], w_ref[...], preferred_element_type=jnp.float32)
        * scale
    )


def _tc2_body(agg_ref, hp_ref, dinv_ref, b_ref, w_ref, out_ref):
    ssum = agg_ref[0] + agg_ref[1] + hp_ref[...]
    scale = dinv_ref[...][:, 0:1]
    h1 = jnp.maximum(ssum * scale + b_ref[...], 0.0)
    out_ref[...] = (
        jnp.dot(h1, w_ref[...], preferred_element_type=jnp.float32) * scale
    )


def _tc3_body(agg_ref, hp_ref, dinv_ref, b_ref, out_ref):
    ssum = agg_ref[0] + agg_ref[1] + hp_ref[...]
    scale = dinv_ref[...][:, 0:1]
    out_ref[...] = jnp.maximum(ssum * scale + b_ref[...], 0.0)


_tc1 = pl.pallas_call(
    _tc1_body,
    grid=(N // _RB,),
    in_specs=[
        pl.BlockSpec((_RB, D), lambda i: (i, 0)),
        pl.BlockSpec((D, D), lambda i: (0, 0)),
        pl.BlockSpec((NC, _RB, DEGW), lambda i: (0, i, 0)),
    ],
    out_specs=[
        pl.BlockSpec((_RB, D), lambda i: (i, 0)),
        pl.BlockSpec((_RB, DEGW), lambda i: (i, 0)),
    ],
    out_shape=[
        jax.ShapeDtypeStruct((N, D), jnp.float32),
        jax.ShapeDtypeStruct((N, DEGW), jnp.float32),
    ],
)

_tc2 = pl.pallas_call(
    _tc2_body,
    grid=(N // _RB,),
    in_specs=[
        pl.BlockSpec((NC, _RB, D), lambda i: (0, i, 0)),
        pl.BlockSpec((_RB, D), lambda i: (i, 0)),
        pl.BlockSpec((_RB, DEGW), lambda i: (i, 0)),
        pl.BlockSpec((1, D), lambda i: (0, 0)),
        pl.BlockSpec((D, D), lambda i: (0, 0)),
    ],
    out_specs=pl.BlockSpec((_RB, D), lambda i: (i, 0)),
    out_shape=jax.ShapeDtypeStruct((N, D), jnp.float32),
)

_tc3 = pl.pallas_call(
    _tc3_body,
    grid=(N // _RB,),
    in_specs=[
        pl.BlockSpec((NC, _RB, D), lambda i: (0, i, 0)),
        pl.BlockSpec((_RB, D), lambda i: (i, 0)),
        pl.BlockSpec((_RB, DEGW), lambda i: (i, 0)),
        pl.BlockSpec((1, D), lambda i: (0, 0)),
    ],
    out_specs=pl.BlockSpec((_RB, D), lambda i: (i, 0)),
    out_shape=jax.ShapeDtypeStruct((N, D), jnp.float32),
)


@jax.jit
def kernel(x, edge_index, W1, b1, W2, b2):
    src = edge_index[0].astype(jnp.int32)
    dst = edge_index[1].astype(jnp.int32)
    pad = EPAD - E
    src_p = jnp.concatenate([src, jnp.zeros((pad,), jnp.int32)])
    dst_p = jnp.concatenate([dst, jnp.full((pad,), N, jnp.int32)])

    degp = _deg_kernel(dst_p)
    hp1, dinv16 = _tc1(x, W1, degp)
    agg1 = _agg_kernel(hp1, src_p, dst_p)
    h2p = _tc2(agg1, hp1, dinv16, b1.reshape(1, D), W2)
    agg2 = _agg_kernel(h2p, src_p, dst_p)
    return _tc3(agg2, h2p, dinv16, b2.reshape(1, D))
